# baseline (speedup 1.0000x reference)
import functools

import jax
import jax.numpy as jnp
from jax import lax
from jax.experimental import pallas as pl
from jax.experimental.pallas import tpu as pltpu
from jax.experimental.pallas import tpu_sc as plsc

_N = 10000
_E = 320000
_F_IN = 128
_HID = 128
_C = 40
_C_PAD = 64

_NC = 2
_NS = 16
_NW = _NC * _NS
_K = 80
_EPW_RAW = _E // _NS
_NBLK = 256
_EPW = _NBLK * _K
_RCHUNK = 80
_NCHUNK = _N // _RCHUNK


def _row_block(i):
    return (i, 0)


def _mm1_body(x_ref, w_ref, o_ref):
    o_ref[...] = lax.dot(x_ref[...], w_ref[0],
                         precision=lax.Precision.HIGHEST,
                         preferred_element_type=jnp.float32)


def _mm2_body(pa_ref, pb_ref, b1_ref, w2_ref, o_ref):
    a = jnp.maximum(jnp.concatenate([pa_ref[...], pb_ref[...]], axis=1)
                    + b1_ref[...], 0.0)
    o_ref[...] = lax.dot(a, w2_ref[0],
                         precision=lax.Precision.HIGHEST,
                         preferred_element_type=jnp.float32)


def _out_body(pa_ref, pb_ref, b2_ref, o_ref):
    z = jnp.concatenate([pa_ref[...], pb_ref[...]], axis=1)[:, :_C] + b2_ref[...]
    m = jnp.max(z, axis=1, keepdims=True)
    s = z - m
    lse = jnp.log(jnp.sum(jnp.exp(s), axis=1, keepdims=True))
    o_ref[...] = s - lse


def _make_spmm(D):
    nv = D // 16
    mesh = plsc.VectorSubcoreMesh(core_axis_name="c", subcore_axis_name="s")

    @functools.partial(
        pl.kernel,
        out_type=jax.ShapeDtypeStruct((_NC * _N, D), jnp.float32),
        mesh=mesh,
        compiler_params=pltpu.CompilerParams(use_tc_tiling_on_sc=False),
        scratch_types=[
            pltpu.VMEM((_NBLK, _K), jnp.int32),
            pltpu.VMEM((_NBLK, _K), jnp.int32),
            pltpu.VMEM((_NBLK, _K), jnp.float32),
            pltpu.VMEM((_K, D), jnp.float32),
            pltpu.VMEM((_K, D), jnp.float32),
            pltpu.VMEM((_K, D), jnp.float32),
            pltpu.VMEM((_K, D), jnp.float32),
            pltpu.VMEM((_RCHUNK, D), jnp.float32),
            pltpu.VMEM_SHARED((_N, D), jnp.float32),
            pltpu.SemaphoreType.DMA,
            pltpu.SemaphoreType.DMA,
            pltpu.SemaphoreType.DMA,
            pltpu.SemaphoreType.DMA,
        ],
    )
    def spmm(h_hbm, col_hbm, row_hbm, w_hbm, out_hbm,
             col_v, row_v, w_v, r0, r1, r2, r3, zbuf, acc_sh,
             g0, g1, g2, g3):
        rows = (r0, r1, r2, r3)
        gsem = (g0, g1, g2, g3)
        cid = lax.axis_index("c")
        sid = lax.axis_index("s")
        wid = cid * _NS + sid

        pltpu.sync_copy(col_hbm.at[wid], col_v)
        pltpu.sync_copy(row_hbm.at[wid], row_v)
        pltpu.sync_copy(w_hbm.at[wid], w_v)

        zeros16 = jnp.zeros((16,), jnp.float32)

        def zrow(i, carry):
            for v in range(nv):
                zbuf[i, pl.ds(v * 16, 16)] = zeros16
            return carry

        lax.fori_loop(0, _RCHUNK, zrow, 0)

        def zchunk(c, carry):
            chunk = c * _NS + sid

            @pl.when(chunk < _NCHUNK)
            def _():
                pltpu.sync_copy(zbuf, acc_sh.at[pl.ds(chunk * _RCHUNK, _RCHUNK)])

            return carry

        lax.fori_loop(0, pl.cdiv(_NCHUNK, _NS), zchunk, 0)
        plsc.subcore_barrier()

        def scale(j, rows_v):
            @plsc.parallel_loop(0, _K, step=16)
            def _(k0):
                w16 = w_v[j, pl.ds(k0, 16)]
                for i in range(16):
                    wi = w16[i]
                    for v in range(nv):
                        sl = pl.ds(v * 16, 16)
                        rows_v[k0 + i, sl] = rows_v[k0 + i, sl] * wi

        for q in range(3):
            pltpu.async_copy(h_hbm.at[col_v.at[q]], rows[q], gsem[q])

        def blk(p, carry):
            for q in range(4):
                b = 4 * p + q
                qn = (q + 3) % 4
                pltpu.make_async_copy(h_hbm.at[col_v.at[b]], rows[q],
                                      gsem[q]).wait()
                scale(b, rows[q])
                pltpu.sync_copy(rows[q], acc_sh.at[row_v.at[b]], add=True)

                @pl.when(b + 3 < _NBLK)
                def _():
                    pltpu.async_copy(h_hbm.at[col_v.at[b + 3]], rows[qn],
                                     gsem[qn])

            return carry

        lax.fori_loop(0, _NBLK // 4, blk, 0)
        plsc.subcore_barrier()

        def ochunk(c, carry):
            chunk = c * _NS + sid

            @pl.when(chunk < _NCHUNK)
            def _():
                off = chunk * _RCHUNK
                pltpu.sync_copy(acc_sh.at[pl.ds(off, _RCHUNK)],
                                out_hbm.at[pl.ds(cid * _N + off, _RCHUNK)])

            return carry

        lax.fori_loop(0, pl.cdiv(_NCHUNK, _NS), ochunk, 0)

    return spmm


_spmm_64 = _make_spmm(_HID // 2)
_spmm_32 = _make_spmm(_C_PAD // 2)

_RB = 1000
_NRB = _N // _RB


@jax.jit
def kernel(x, edge_index, edge_weight, W1, b1, W2, b2):
    pad_i = jnp.zeros((_NS, _EPW - _EPW_RAW), jnp.int32)
    pad_f = jnp.zeros((_NS, _EPW - _EPW_RAW), jnp.float32)
    rowc = jnp.concatenate(
        [edge_index[0].reshape(_NS, _EPW_RAW), pad_i], 1).reshape(1, _NS, _NBLK, _K)
    colc = jnp.concatenate(
        [edge_index[1].reshape(_NS, _EPW_RAW), pad_i], 1).reshape(1, _NS, _NBLK, _K)
    wc = jnp.concatenate(
        [edge_weight.reshape(_NS, _EPW_RAW), pad_f], 1).reshape(1, _NS, _NBLK, _K)
    col3 = jnp.concatenate([colc, colc + _N], 0).reshape(_NW, _NBLK, _K)
    row3 = jnp.broadcast_to(rowc, (_NC, _NS, _NBLK, _K)).reshape(_NW, _NBLK, _K)
    w3 = jnp.broadcast_to(wc, (_NC, _NS, _NBLK, _K)).reshape(_NW, _NBLK, _K)
    W2p = jnp.pad(W2, ((0, 0), (0, _C_PAD - _C)))
    W1h = jnp.stack([W1[:, :_HID // 2], W1[:, _HID // 2:]])
    W2h = jnp.stack([W2p[:, :_C_PAD // 2], W2p[:, _C_PAD // 2:]])
    b1r = b1.reshape(1, _HID)
    b2r = b2.reshape(1, _C)

    h1s = pl.pallas_call(
        _mm1_body,
        grid=(2, _NRB),
        in_specs=[pl.BlockSpec((_RB, _F_IN), lambda j, i: (i, 0)),
                  pl.BlockSpec((1, _F_IN, _HID // 2), lambda j, i: (j, 0, 0))],
        out_specs=pl.BlockSpec((_RB, _HID // 2), lambda j, i: (j * _NRB + i, 0)),
        out_shape=jax.ShapeDtypeStruct((_NC * _N, _HID // 2), jnp.float32),
    )(x, W1h)

    p1 = _spmm_64(h1s, col3, row3, w3)

    h2s = pl.pallas_call(
        _mm2_body,
        grid=(2, _NRB),
        in_specs=[pl.BlockSpec((_RB, _HID // 2), lambda j, i: (i, 0)),
                  pl.BlockSpec((_RB, _HID // 2), lambda j, i: (i + _NRB, 0)),
                  pl.BlockSpec((1, _HID), lambda j, i: (0, 0)),
                  pl.BlockSpec((1, _HID, _C_PAD // 2), lambda j, i: (j, 0, 0))],
        out_specs=pl.BlockSpec((_RB, _C_PAD // 2), lambda j, i: (j * _NRB + i, 0)),
        out_shape=jax.ShapeDtypeStruct((_NC * _N, _C_PAD // 2), jnp.float32),
    )(p1, p1, b1r, W2h)

    p2 = _spmm_32(h2s, col3, row3, w3)

    out = pl.pallas_call(
        _out_body,
        grid=(_NRB,),
        in_specs=[pl.BlockSpec((_RB, _C_PAD // 2), _row_block),
                  pl.BlockSpec((_RB, _C_PAD // 2), lambda i: (i + _NRB, 0)),
                  pl.BlockSpec((1, _C), lambda i: (0, 0))],
        out_specs=pl.BlockSpec((_RB, _C), _row_block),
        out_shape=jax.ShapeDtypeStruct((_N, _C), jnp.float32),
    )(p2, p2, b2r)

    return out


# speedup vs baseline: 1.2883x; 1.2883x over previous
import functools

import jax
import jax.numpy as jnp
from jax import lax
from jax.experimental import pallas as pl
from jax.experimental.pallas import tpu as pltpu
from jax.experimental.pallas import tpu_sc as plsc

_N = 10000
_E = 320000
_F_IN = 128
_HID = 128
_C = 40
_C_PAD = 64

_NC = 2
_NS = 16
_NW = _NC * _NS
_K = 80
_EPW = _E // _NS
_NBLK = _EPW // _K
_RCHUNK = 80
_NCHUNK = _N // _RCHUNK


def _row_block(i):
    return (i, 0)


def _mm1_body(x_ref, w_ref, o_ref):
    o_ref[...] = lax.dot(x_ref[...], w_ref[0],
                         precision=lax.Precision.HIGHEST,
                         preferred_element_type=jnp.float32)


def _mm2_body(pa_ref, pb_ref, b1_ref, w2_ref, o_ref):
    a = jnp.maximum(jnp.concatenate([pa_ref[...], pb_ref[...]], axis=1)
                    + b1_ref[...], 0.0)
    o_ref[...] = lax.dot(a, w2_ref[0],
                         precision=lax.Precision.HIGHEST,
                         preferred_element_type=jnp.float32)


def _out_body(pa_ref, pb_ref, b2_ref, o_ref):
    z = jnp.concatenate([pa_ref[...], pb_ref[...]], axis=1)[:, :_C] + b2_ref[...]
    m = jnp.max(z, axis=1, keepdims=True)
    s = z - m
    lse = jnp.log(jnp.sum(jnp.exp(s), axis=1, keepdims=True))
    o_ref[...] = s - lse


def _make_spmm(D):
    nv = D // 16
    mesh = plsc.VectorSubcoreMesh(core_axis_name="c", subcore_axis_name="s")

    @functools.partial(
        pl.kernel,
        out_type=jax.ShapeDtypeStruct((_NC * _N, D), jnp.float32),
        mesh=mesh,
        compiler_params=pltpu.CompilerParams(use_tc_tiling_on_sc=False),
        scratch_types=[
            pltpu.VMEM((_NBLK, _K), jnp.int32),
            pltpu.VMEM((_NBLK, _K), jnp.int32),
            pltpu.VMEM((_NBLK, _K), jnp.float32),
            pltpu.VMEM((_K, D), jnp.float32),
            pltpu.VMEM((_K, D), jnp.float32),
            pltpu.VMEM((_RCHUNK, D), jnp.float32),
            pltpu.VMEM_SHARED((_N, D), jnp.float32),
            pltpu.SemaphoreType.DMA,
            pltpu.SemaphoreType.DMA,
        ],
    )
    def spmm(h_hbm, col_hbm, row_hbm, w_hbm, out_hbm,
             col_v, row_v, w_v, rows_a, rows_b, zbuf, acc_sh, sem_a, sem_b):
        cid = lax.axis_index("c")
        sid = lax.axis_index("s")
        wid = cid * _NS + sid

        pltpu.sync_copy(col_hbm.at[wid], col_v)
        pltpu.sync_copy(row_hbm.at[wid], row_v)
        pltpu.sync_copy(w_hbm.at[wid], w_v)

        zeros16 = jnp.zeros((16,), jnp.float32)

        def zrow(i, carry):
            for v in range(nv):
                zbuf[i, pl.ds(v * 16, 16)] = zeros16
            return carry

        lax.fori_loop(0, _RCHUNK, zrow, 0)

        def zchunk(c, carry):
            chunk = c * _NS + sid

            @pl.when(chunk < _NCHUNK)
            def _():
                pltpu.sync_copy(zbuf, acc_sh.at[pl.ds(chunk * _RCHUNK, _RCHUNK)])

            return carry

        lax.fori_loop(0, pl.cdiv(_NCHUNK, _NS), zchunk, 0)
        plsc.subcore_barrier()

        def scale_scatter(j, rows_v):
            @plsc.parallel_loop(0, _K, step=16)
            def _(k0):
                w16 = w_v[j, pl.ds(k0, 16)]
                for i in range(16):
                    wi = w16[i]
                    for v in range(nv):
                        sl = pl.ds(v * 16, 16)
                        rows_v[k0 + i, sl] = rows_v[k0 + i, sl] * wi

            pltpu.sync_copy(rows_v, acc_sh.at[row_v.at[j]], add=True)

        pltpu.async_copy(h_hbm.at[col_v.at[0]], rows_a, sem_a)

        def blk(p, carry):
            b0 = 2 * p
            b1 = b0 + 1
            pltpu.async_copy(h_hbm.at[col_v.at[b1]], rows_b, sem_b)
            pltpu.make_async_copy(h_hbm.at[col_v.at[b0]], rows_a, sem_a).wait()
            scale_scatter(b0, rows_a)

            @pl.when(b1 + 1 < _NBLK)
            def _():
                pltpu.async_copy(h_hbm.at[col_v.at[b1 + 1]], rows_a, sem_a)

            pltpu.make_async_copy(h_hbm.at[col_v.at[b1]], rows_b, sem_b).wait()
            scale_scatter(b1, rows_b)
            return carry

        lax.fori_loop(0, _NBLK // 2, blk, 0)
        plsc.subcore_barrier()

        def ochunk(c, carry):
            chunk = c * _NS + sid

            @pl.when(chunk < _NCHUNK)
            def _():
                off = chunk * _RCHUNK
                pltpu.sync_copy(acc_sh.at[pl.ds(off, _RCHUNK)],
                                out_hbm.at[pl.ds(cid * _N + off, _RCHUNK)])

            return carry

        lax.fori_loop(0, pl.cdiv(_NCHUNK, _NS), ochunk, 0)

    return spmm


_spmm_64 = _make_spmm(_HID // 2)
_spmm_32 = _make_spmm(_C_PAD // 2)

_RB = 1000
_NRB = _N // _RB


@jax.jit
def kernel(x, edge_index, edge_weight, W1, b1, W2, b2):
    rowc = edge_index[0].reshape(1, _NS, _NBLK, _K)
    colc = edge_index[1].reshape(1, _NS, _NBLK, _K)
    wc = edge_weight.reshape(1, _NS, _NBLK, _K)
    col3 = jnp.concatenate([colc, colc + _N], 0).reshape(_NW, _NBLK, _K)
    row3 = jnp.broadcast_to(rowc, (_NC, _NS, _NBLK, _K)).reshape(_NW, _NBLK, _K)
    w3 = jnp.broadcast_to(wc, (_NC, _NS, _NBLK, _K)).reshape(_NW, _NBLK, _K)
    W2p = jnp.pad(W2, ((0, 0), (0, _C_PAD - _C)))
    W1h = jnp.stack([W1[:, :_HID // 2], W1[:, _HID // 2:]])
    W2h = jnp.stack([W2p[:, :_C_PAD // 2], W2p[:, _C_PAD // 2:]])
    b1r = b1.reshape(1, _HID)
    b2r = b2.reshape(1, _C)

    h1s = pl.pallas_call(
        _mm1_body,
        grid=(2, _NRB),
        in_specs=[pl.BlockSpec((_RB, _F_IN), lambda j, i: (i, 0)),
                  pl.BlockSpec((1, _F_IN, _HID // 2), lambda j, i: (j, 0, 0))],
        out_specs=pl.BlockSpec((_RB, _HID // 2), lambda j, i: (j * _NRB + i, 0)),
        out_shape=jax.ShapeDtypeStruct((_NC * _N, _HID // 2), jnp.float32),
    )(x, W1h)

    p1 = _spmm_64(h1s, col3, row3, w3)

    h2s = pl.pallas_call(
        _mm2_body,
        grid=(2, _NRB),
        in_specs=[pl.BlockSpec((_RB, _HID // 2), lambda j, i: (i, 0)),
                  pl.BlockSpec((_RB, _HID // 2), lambda j, i: (i + _NRB, 0)),
                  pl.BlockSpec((1, _HID), lambda j, i: (0, 0)),
                  pl.BlockSpec((1, _HID, _C_PAD // 2), lambda j, i: (j, 0, 0))],
        out_specs=pl.BlockSpec((_RB, _C_PAD // 2), lambda j, i: (j * _NRB + i, 0)),
        out_shape=jax.ShapeDtypeStruct((_NC * _N, _C_PAD // 2), jnp.float32),
    )(p1, p1, b1r, W2h)

    p2 = _spmm_32(h2s, col3, row3, w3)

    out = pl.pallas_call(
        _out_body,
        grid=(_NRB,),
        in_specs=[pl.BlockSpec((_RB, _C_PAD // 2), _row_block),
                  pl.BlockSpec((_RB, _C_PAD // 2), lambda i: (i + _NRB, 0)),
                  pl.BlockSpec((1, _C), lambda i: (0, 0))],
        out_specs=pl.BlockSpec((_RB, _C), _row_block),
        out_shape=jax.ShapeDtypeStruct((_N, _C), jnp.float32),
    )(p2, p2, b2r)

    return out


# speedup vs baseline: 1.3275x; 1.0304x over previous
import functools

import jax
import jax.numpy as jnp
from jax import lax
from jax.experimental import pallas as pl
from jax.experimental.pallas import tpu as pltpu
from jax.experimental.pallas import tpu_sc as plsc

_N = 10000
_E = 320000
_F_IN = 128
_HID = 128
_C = 40
_C_PAD = 64

_NC = 2
_NS = 16
_NW = _NC * _NS
_K = 80
_EPW = _E // _NS
_NBLK = _EPW // _K
_RCHUNK = 80
_NCHUNK = _N // _RCHUNK


def _row_block(i):
    return (i, 0)


def _mm1_body(x_ref, w_ref, o_ref):
    o_ref[...] = lax.dot(x_ref[...], w_ref[0],
                         precision=lax.Precision.DEFAULT,
                         preferred_element_type=jnp.float32)


def _mm2_body(pa_ref, pb_ref, b1_ref, w2_ref, o_ref):
    a = jnp.maximum(jnp.concatenate([pa_ref[...], pb_ref[...]], axis=1)
                    + b1_ref[...], 0.0)
    o_ref[...] = lax.dot(a, w2_ref[0],
                         precision=lax.Precision.DEFAULT,
                         preferred_element_type=jnp.float32)


def _out_body(pa_ref, pb_ref, b2_ref, o_ref):
    z = jnp.concatenate([pa_ref[...], pb_ref[...]], axis=1)[:, :_C] + b2_ref[...]
    m = jnp.max(z, axis=1, keepdims=True)
    s = z - m
    lse = jnp.log(jnp.sum(jnp.exp(s), axis=1, keepdims=True))
    o_ref[...] = s - lse


def _make_spmm(D):
    nv = D // 16
    mesh = plsc.VectorSubcoreMesh(core_axis_name="c", subcore_axis_name="s")

    @functools.partial(
        pl.kernel,
        out_type=jax.ShapeDtypeStruct((_NC * _N, D), jnp.float32),
        mesh=mesh,
        compiler_params=pltpu.CompilerParams(use_tc_tiling_on_sc=False),
        scratch_types=[
            pltpu.VMEM((_NBLK, _K), jnp.int32),
            pltpu.VMEM((_NBLK, _K), jnp.int32),
            pltpu.VMEM((_NBLK, _K), jnp.float32),
            pltpu.VMEM((_K, D), jnp.float32),
            pltpu.VMEM((_K, D), jnp.float32),
            pltpu.VMEM((_RCHUNK, D), jnp.float32),
            pltpu.VMEM_SHARED((_N, D), jnp.float32),
            pltpu.SemaphoreType.DMA,
            pltpu.SemaphoreType.DMA,
        ],
    )
    def spmm(h_hbm, col_hbm, row_hbm, w_hbm, out_hbm,
             col_v, row_v, w_v, rows_a, rows_b, zbuf, acc_sh, sem_a, sem_b):
        cid = lax.axis_index("c")
        sid = lax.axis_index("s")
        wid = cid * _NS + sid

        pltpu.sync_copy(col_hbm.at[wid], col_v)
        pltpu.sync_copy(row_hbm.at[wid], row_v)
        pltpu.sync_copy(w_hbm.at[wid], w_v)

        zeros16 = jnp.zeros((16,), jnp.float32)

        def zrow(i, carry):
            for v in range(nv):
                zbuf[i, pl.ds(v * 16, 16)] = zeros16
            return carry

        lax.fori_loop(0, _RCHUNK, zrow, 0)

        def zchunk(c, carry):
            chunk = c * _NS + sid

            @pl.when(chunk < _NCHUNK)
            def _():
                pltpu.sync_copy(zbuf, acc_sh.at[pl.ds(chunk * _RCHUNK, _RCHUNK)])

            return carry

        lax.fori_loop(0, pl.cdiv(_NCHUNK, _NS), zchunk, 0)
        plsc.subcore_barrier()

        def scale_scatter(j, rows_v):
            @plsc.parallel_loop(0, _K, step=16)
            def _(k0):
                w16 = w_v[j, pl.ds(k0, 16)]
                for i in range(16):
                    wi = w16[i]
                    for v in range(nv):
                        sl = pl.ds(v * 16, 16)
                        rows_v[k0 + i, sl] = rows_v[k0 + i, sl] * wi

            pltpu.sync_copy(rows_v, acc_sh.at[row_v.at[j]], add=True)

        pltpu.async_copy(h_hbm.at[col_v.at[0]], rows_a, sem_a)

        def blk(p, carry):
            b0 = 2 * p
            b1 = b0 + 1
            pltpu.async_copy(h_hbm.at[col_v.at[b1]], rows_b, sem_b)
            pltpu.make_async_copy(h_hbm.at[col_v.at[b0]], rows_a, sem_a).wait()
            scale_scatter(b0, rows_a)

            @pl.when(b1 + 1 < _NBLK)
            def _():
                pltpu.async_copy(h_hbm.at[col_v.at[b1 + 1]], rows_a, sem_a)

            pltpu.make_async_copy(h_hbm.at[col_v.at[b1]], rows_b, sem_b).wait()
            scale_scatter(b1, rows_b)
            return carry

        lax.fori_loop(0, _NBLK // 2, blk, 0)
        plsc.subcore_barrier()

        def ochunk(c, carry):
            chunk = c * _NS + sid

            @pl.when(chunk < _NCHUNK)
            def _():
                off = chunk * _RCHUNK
                pltpu.sync_copy(acc_sh.at[pl.ds(off, _RCHUNK)],
                                out_hbm.at[pl.ds(cid * _N + off, _RCHUNK)])

            return carry

        lax.fori_loop(0, pl.cdiv(_NCHUNK, _NS), ochunk, 0)

    return spmm


_spmm_64 = _make_spmm(_HID // 2)
_spmm_32 = _make_spmm(_C_PAD // 2)

_RB = 1000
_NRB = _N // _RB


@jax.jit
def kernel(x, edge_index, edge_weight, W1, b1, W2, b2):
    rowc = edge_index[0].reshape(1, _NS, _NBLK, _K)
    colc = edge_index[1].reshape(1, _NS, _NBLK, _K)
    wc = edge_weight.reshape(1, _NS, _NBLK, _K)
    col3 = jnp.concatenate([colc, colc + _N], 0).reshape(_NW, _NBLK, _K)
    row3 = jnp.broadcast_to(rowc, (_NC, _NS, _NBLK, _K)).reshape(_NW, _NBLK, _K)
    w3 = jnp.broadcast_to(wc, (_NC, _NS, _NBLK, _K)).reshape(_NW, _NBLK, _K)
    W2p = jnp.pad(W2, ((0, 0), (0, _C_PAD - _C)))
    W1h = jnp.stack([W1[:, :_HID // 2], W1[:, _HID // 2:]])
    W2h = jnp.stack([W2p[:, :_C_PAD // 2], W2p[:, _C_PAD // 2:]])
    b1r = b1.reshape(1, _HID)
    b2r = b2.reshape(1, _C)

    h1s = pl.pallas_call(
        _mm1_body,
        grid=(2, _NRB),
        in_specs=[pl.BlockSpec((_RB, _F_IN), lambda j, i: (i, 0)),
                  pl.BlockSpec((1, _F_IN, _HID // 2), lambda j, i: (j, 0, 0))],
        out_specs=pl.BlockSpec((_RB, _HID // 2), lambda j, i: (j * _NRB + i, 0)),
        out_shape=jax.ShapeDtypeStruct((_NC * _N, _HID // 2), jnp.float32),
    )(x, W1h)

    p1 = _spmm_64(h1s, col3, row3, w3)

    h2s = pl.pallas_call(
        _mm2_body,
        grid=(2, _NRB),
        in_specs=[pl.BlockSpec((_RB, _HID // 2), lambda j, i: (i, 0)),
                  pl.BlockSpec((_RB, _HID // 2), lambda j, i: (i + _NRB, 0)),
                  pl.BlockSpec((1, _HID), lambda j, i: (0, 0)),
                  pl.BlockSpec((1, _HID, _C_PAD // 2), lambda j, i: (j, 0, 0))],
        out_specs=pl.BlockSpec((_RB, _C_PAD // 2), lambda j, i: (j * _NRB + i, 0)),
        out_shape=jax.ShapeDtypeStruct((_NC * _N, _C_PAD // 2), jnp.float32),
    )(p1, p1, b1r, W2h)

    p2 = _spmm_32(h2s, col3, row3, w3)

    out = pl.pallas_call(
        _out_body,
        grid=(_NRB,),
        in_specs=[pl.BlockSpec((_RB, _C_PAD // 2), _row_block),
                  pl.BlockSpec((_RB, _C_PAD // 2), lambda i: (i + _NRB, 0)),
                  pl.BlockSpec((1, _C), lambda i: (0, 0))],
        out_specs=pl.BlockSpec((_RB, _C), _row_block),
        out_shape=jax.ShapeDtypeStruct((_N, _C), jnp.float32),
    )(p2, p2, b2r)

    return out
